# baseline (speedup 1.0000x reference)
import functools

import jax
import jax.numpy as jnp
from jax import lax
from jax.experimental import pallas as pl
from jax.experimental.pallas import tpu as pltpu
from jax.experimental.pallas import tpu_sc as plsc

S, U, H, V = 20, 256, 128, 10000
LAMBDA_LOC = 0.1


@functools.cache
def _make_sc_gather():
    info = plsc.get_sparse_core_info()
    nc, ns = info.num_cores, info.num_subcores
    nw = nc * ns
    xw = (S * U) // nw
    uw = U // nw
    xc = xw // 2
    mesh = plsc.VectorSubcoreMesh(core_axis_name="c", subcore_axis_name="s")

    @functools.partial(
        pl.kernel,
        out_type=(jax.ShapeDtypeStruct((S * U, H), jnp.float32),
                  jax.ShapeDtypeStruct((U, H), jnp.float32)),
        mesh=mesh,
        scratch_types=[
            pltpu.VMEM((xw,), jnp.int32),
            pltpu.VMEM((uw,), jnp.int32),
            pltpu.VMEM((xw, H), jnp.float32),
            pltpu.VMEM((uw, H), jnp.float32),
            pltpu.SemaphoreType.DMA,
        ],
    )
    def gather_k(table_hbm, xidx_hbm, uidx_hbm, outx_hbm, outu_hbm,
                 xidx_v, uidx_v, xrows_v, urows_v, sem):
        wid = lax.axis_index("s") * nc + lax.axis_index("c")
        xbase = wid * xw
        ubase = wid * uw
        pltpu.sync_copy(xidx_hbm.at[pl.ds(xbase, xw)], xidx_v)
        pltpu.sync_copy(uidx_hbm.at[pl.ds(ubase, uw)], uidx_v)
        copies = [
            pltpu.async_copy(table_hbm.at[xidx_v.at[pl.ds(k * xc, xc)]],
                             xrows_v.at[pl.ds(k * xc, xc)], sem)
            for k in range(2)
        ]
        copies.append(pltpu.async_copy(table_hbm.at[uidx_v], urows_v, sem))
        for c in copies:
            c.wait()
        pltpu.sync_copy(xrows_v, outx_hbm.at[pl.ds(xbase, xw)])
        pltpu.sync_copy(urows_v, outu_hbm.at[pl.ds(ubase, uw)])

    return gather_k


def _fuse_body(xemb_ref, pu_ref, t_ref, sx_ref, sy_ref, h0_ref, wih_ref,
               whh_ref, bih_ref, bhh_ref, wf_ref, bf_ref, out_ref,
               hs_ref, t2v_ref):
    hi = lax.Precision.HIGHEST

    def gru_step(i, hprev):
        xt = xemb_ref[i]
        gi = lax.dot_general(xt, wih_ref[...], (((1,), (1,)), ((), ())),
                             precision=hi,
                             preferred_element_type=jnp.float32) + bih_ref[...]
        gh = lax.dot_general(hprev, whh_ref[...], (((1,), (1,)), ((), ())),
                             precision=hi,
                             preferred_element_type=jnp.float32) + bhh_ref[...]
        r = jax.nn.sigmoid(gi[:, :H] + gh[:, :H])
        z = jax.nn.sigmoid(gi[:, H:2 * H] + gh[:, H:2 * H])
        n = jnp.tanh(gi[:, 2 * H:] + r * gh[:, 2 * H:])
        hn = (1.0 - z) * n + z * hprev
        hs_ref[pl.ds(i, 1)] = hn[None]
        return hn

    lax.fori_loop(0, S, gru_step, h0_ref[...], unroll=True)

    tau = t_ref[...][:, :, None]
    ph = tau * wf_ref[...][0] + bf_ref[...][0]
    ch = lax.broadcasted_iota(jnp.int32, (1, 1, H), 2)
    t2v_ref[...] = jnp.where(ch == 0, ph, jnp.sin(ph))

    sx = sx_ref[...]
    sy = sy_ref[...]
    jidx = lax.broadcasted_iota(jnp.int32, (S, U), 0)

    def w_step(i, _):
        d = t2v_ref[...] - t2v_ref[i][None]
        asq = jnp.sum(d * d, axis=2)
        dx = sx - sx_ref[pl.ds(i, 1)]
        dy = sy - sy_ref[pl.ds(i, 1)]
        dist = jnp.sqrt(dx * dx + dy * dy + 1e-12)
        w = jnp.exp(-(asq + LAMBDA_LOC * dist)) + 1e-10
        w = jnp.where(jidx <= i, w, 0.0)
        sw = jnp.sum(w, axis=0)
        acc = jnp.sum(w[:, :, None] * hs_ref[...], axis=0)
        out_ref[pl.ds(i, 1), :, :H] = (acc / sw[:, None])[None]
        return 0

    lax.fori_loop(0, S, w_step, 0)
    out_ref[:, :, H:] = jnp.broadcast_to(pu_ref[...][None], (S, U, H))


def _mm_body(a_ref, w_ref, b_ref, o_ref):
    a = a_ref[...].reshape(_BS * U, 2 * H).astype(jnp.bfloat16)
    w = w_ref[...].astype(jnp.bfloat16)
    y = lax.dot_general(a, w, (((1,), (1,)), ((), ())),
                        preferred_element_type=jnp.float32) + b_ref[...]
    o_ref[...] = y.reshape(_BS, U, _BN)


_BS, _BN = 5, 1024


def _fc_matmul(a, fc_W, fc_b2):
    nm = S // _BS
    nn = pl.cdiv(V, _BN)
    return pl.pallas_call(
        _mm_body,
        grid=(nm, nn),
        in_specs=[
            pl.BlockSpec((_BS, U, 2 * H), lambda m, n: (m, 0, 0)),
            pl.BlockSpec((_BN, 2 * H), lambda m, n: (n, 0)),
            pl.BlockSpec((1, _BN), lambda m, n: (0, n)),
        ],
        out_specs=pl.BlockSpec((_BS, U, _BN), lambda m, n: (m, 0, n)),
        out_shape=jax.ShapeDtypeStruct((S, U, V), jnp.float32),
    )(a, fc_W, fc_b2)


def kernel(x, t, s, h, active_user, enc_table, gru_Wih, gru_Whh, gru_bih,
           gru_bhh, t2v_w0, t2v_b0, t2v_W, t2v_B, fc_W, fc_b):
    x_rows, u_rows = _make_sc_gather()(
        enc_table, x.reshape(-1).astype(jnp.int32),
        active_user.reshape(-1).astype(jnp.int32))
    x_emb = x_rows.reshape(S, U, H)

    wf = jnp.concatenate([t2v_w0, t2v_W[0]]).reshape(1, H)
    bf = jnp.concatenate([t2v_b0, t2v_B]).reshape(1, H)
    out_pu = pl.pallas_call(
        _fuse_body,
        out_shape=jax.ShapeDtypeStruct((S, U, 2 * H), jnp.float32),
        scratch_shapes=[
            pltpu.VMEM((S, U, H), jnp.float32),
            pltpu.VMEM((S, U, H), jnp.float32),
        ],
    )(x_emb, u_rows, t, s[:, :, 0], s[:, :, 1], h[0], gru_Wih, gru_Whh,
      gru_bih.reshape(1, 3 * H), gru_bhh.reshape(1, 3 * H), wf, bf)

    return _fc_matmul(out_pu, fc_W, fc_b.reshape(1, V))


# speedup vs baseline: 1.0534x; 1.0534x over previous
import functools

import jax
import jax.numpy as jnp
from jax import lax
from jax.experimental import pallas as pl
from jax.experimental.pallas import tpu as pltpu
from jax.experimental.pallas import tpu_sc as plsc

S, U, H, V = 20, 256, 128, 10000
LAMBDA_LOC = 0.1


@functools.cache
def _make_sc_gather():
    info = plsc.get_sparse_core_info()
    nc, ns = info.num_cores, info.num_subcores
    nw = nc * ns
    xw = (S * U) // nw
    uw = U // nw
    xc = xw // 2
    mesh = plsc.VectorSubcoreMesh(core_axis_name="c", subcore_axis_name="s")

    @functools.partial(
        pl.kernel,
        out_type=(jax.ShapeDtypeStruct((S * U, H), jnp.float32),
                  jax.ShapeDtypeStruct((U, H), jnp.float32)),
        mesh=mesh,
        scratch_types=[
            pltpu.VMEM((xw,), jnp.int32),
            pltpu.VMEM((uw,), jnp.int32),
            pltpu.VMEM((xw, H), jnp.float32),
            pltpu.VMEM((uw, H), jnp.float32),
            pltpu.SemaphoreType.DMA,
        ],
    )
    def gather_k(table_hbm, xidx_hbm, uidx_hbm, outx_hbm, outu_hbm,
                 xidx_v, uidx_v, xrows_v, urows_v, sem):
        wid = lax.axis_index("s") * nc + lax.axis_index("c")
        xbase = wid * xw
        ubase = wid * uw
        pltpu.sync_copy(xidx_hbm.at[pl.ds(xbase, xw)], xidx_v)
        pltpu.sync_copy(uidx_hbm.at[pl.ds(ubase, uw)], uidx_v)
        copies = [
            pltpu.async_copy(table_hbm.at[xidx_v.at[pl.ds(k * xc, xc)]],
                             xrows_v.at[pl.ds(k * xc, xc)], sem)
            for k in range(2)
        ]
        copies.append(pltpu.async_copy(table_hbm.at[uidx_v], urows_v, sem))
        for c in copies:
            c.wait()
        pltpu.sync_copy(xrows_v, outx_hbm.at[pl.ds(xbase, xw)])
        pltpu.sync_copy(urows_v, outu_hbm.at[pl.ds(ubase, uw)])

    return gather_k


def _fuse_body(xemb_ref, pu_ref, t_ref, sx_ref, sy_ref, h0_ref, wih_ref,
               whh_ref, bih_ref, bhh_ref, wf_ref, bf_ref, out_ref,
               hs_ref, t2v_ref, gi_ref):
    bf16 = jnp.bfloat16

    xe = xemb_ref[...].reshape(S * U, H).astype(bf16)
    gi_all = lax.dot_general(xe, wih_ref[...].astype(bf16),
                             (((1,), (1,)), ((), ())),
                             preferred_element_type=jnp.float32)
    gi_ref[...] = gi_all.reshape(S, U, 3 * H) + bih_ref[...]

    whh = whh_ref[...].astype(bf16)
    bhh = bhh_ref[...]

    def gru_step(i, hprev):
        gi = gi_ref[i]
        gh = lax.dot_general(hprev.astype(bf16), whh,
                             (((1,), (1,)), ((), ())),
                             preferred_element_type=jnp.float32) + bhh
        r = jax.nn.sigmoid(gi[:, :H] + gh[:, :H])
        z = jax.nn.sigmoid(gi[:, H:2 * H] + gh[:, H:2 * H])
        n = jnp.tanh(gi[:, 2 * H:] + r * gh[:, 2 * H:])
        hn = (1.0 - z) * n + z * hprev
        hs_ref[pl.ds(i, 1)] = hn[None]
        return hn

    lax.fori_loop(0, S, gru_step, h0_ref[...], unroll=True)

    tau = t_ref[...][:, :, None]
    ph = tau * wf_ref[...][0] + bf_ref[...][0]
    ch = lax.broadcasted_iota(jnp.int32, (1, 1, H), 2)
    t2v_ref[...] = jnp.where(ch == 0, ph, jnp.sin(ph))

    for i in range(S):
        t2vi = t2v_ref[i]
        sxi = sx_ref[pl.ds(i, 1)]
        syi = sy_ref[pl.ds(i, 1)]

        def j_body(j, carry, t2vi=t2vi, sxi=sxi, syi=syi):
            acc, sw = carry
            d = t2v_ref[j] - t2vi
            asq = jnp.sum(d * d, axis=1)
            dx = sx_ref[pl.ds(j, 1)] - sxi
            dy = sy_ref[pl.ds(j, 1)] - syi
            dist = jnp.sqrt(dx * dx + dy * dy + 1e-12)[0]
            w = jnp.exp(-(asq + LAMBDA_LOC * dist)) + 1e-10
            return (acc + w[:, None] * hs_ref[j], sw + w)

        acc, sw = lax.fori_loop(
            0, i + 1, j_body,
            (jnp.zeros((U, H), jnp.float32), jnp.zeros((U,), jnp.float32)))
        out_ref[pl.ds(i, 1), :, :H] = (acc / sw[:, None]).astype(bf16)[None]

    out_ref[:, :, H:] = jnp.broadcast_to(
        pu_ref[...].astype(bf16)[None], (S, U, H))


_BN = 512


def _mm_body(a_ref, w_ref, b_ref, o_ref):
    y = lax.dot_general(a_ref[...], w_ref[...].astype(jnp.bfloat16),
                        (((1,), (1,)), ((), ())),
                        preferred_element_type=jnp.float32) + b_ref[...]
    o_ref[...] = y.reshape(S, U, _BN)


def _fc_matmul(a, fc_W, fc_b2):
    return pl.pallas_call(
        _mm_body,
        grid=(pl.cdiv(V, _BN),),
        in_specs=[
            pl.BlockSpec((S * U, 2 * H), lambda n: (0, 0)),
            pl.BlockSpec((_BN, 2 * H), lambda n: (n, 0)),
            pl.BlockSpec((1, _BN), lambda n: (0, n)),
        ],
        out_specs=pl.BlockSpec((S, U, _BN), lambda n: (0, 0, n)),
        out_shape=jax.ShapeDtypeStruct((S, U, V), jnp.float32),
    )(a, fc_W, fc_b2)


def kernel(x, t, s, h, active_user, enc_table, gru_Wih, gru_Whh, gru_bih,
           gru_bhh, t2v_w0, t2v_b0, t2v_W, t2v_B, fc_W, fc_b):
    x_rows, u_rows = _make_sc_gather()(
        enc_table, x.reshape(-1).astype(jnp.int32),
        active_user.reshape(-1).astype(jnp.int32))
    x_emb = x_rows.reshape(S, U, H)

    wf = jnp.concatenate([t2v_w0, t2v_W[0]]).reshape(1, H)
    bf = jnp.concatenate([t2v_b0, t2v_B]).reshape(1, H)
    out_pu = pl.pallas_call(
        _fuse_body,
        out_shape=jax.ShapeDtypeStruct((S, U, 2 * H), jnp.bfloat16),
        scratch_shapes=[
            pltpu.VMEM((S, U, H), jnp.float32),
            pltpu.VMEM((S, U, H), jnp.float32),
            pltpu.VMEM((S, U, 3 * H), jnp.float32),
        ],
    )(x_emb, u_rows, t, s[:, :, 0], s[:, :, 1], h[0], gru_Wih, gru_Whh,
      gru_bih.reshape(1, 3 * H), gru_bhh.reshape(1, 3 * H), wf, bf)

    return _fc_matmul(out_pu.reshape(S * U, 2 * H), fc_W, fc_b.reshape(1, V))


# speedup vs baseline: 2.0990x; 1.9926x over previous
import functools

import jax
import jax.numpy as jnp
from jax import lax
from jax.experimental import pallas as pl
from jax.experimental.pallas import tpu as pltpu
from jax.experimental.pallas import tpu_sc as plsc

S, U, H, V = 20, 256, 128, 10000
LAMBDA_LOC = 0.1


@functools.cache
def _make_sc_gather():
    info = plsc.get_sparse_core_info()
    nc, ns = info.num_cores, info.num_subcores
    nw = nc * ns
    xw = (S * U) // nw
    uw = U // nw
    xc = xw // 2
    mesh = plsc.VectorSubcoreMesh(core_axis_name="c", subcore_axis_name="s")

    @functools.partial(
        pl.kernel,
        out_type=(jax.ShapeDtypeStruct((S * U, H), jnp.float32),
                  jax.ShapeDtypeStruct((U, H), jnp.float32)),
        mesh=mesh,
        scratch_types=[
            pltpu.VMEM((xw,), jnp.int32),
            pltpu.VMEM((uw,), jnp.int32),
            pltpu.VMEM((xw, H), jnp.float32),
            pltpu.VMEM((uw, H), jnp.float32),
            pltpu.SemaphoreType.DMA,
        ],
    )
    def gather_k(table_hbm, xidx_hbm, uidx_hbm, outx_hbm, outu_hbm,
                 xidx_v, uidx_v, xrows_v, urows_v, sem):
        wid = lax.axis_index("s") * nc + lax.axis_index("c")
        xbase = wid * xw
        ubase = wid * uw
        pltpu.sync_copy(xidx_hbm.at[pl.ds(xbase, xw)], xidx_v)
        pltpu.sync_copy(uidx_hbm.at[pl.ds(ubase, uw)], uidx_v)
        copies = [
            pltpu.async_copy(table_hbm.at[xidx_v.at[pl.ds(k * xc, xc)]],
                             xrows_v.at[pl.ds(k * xc, xc)], sem)
            for k in range(2)
        ]
        copies.append(pltpu.async_copy(table_hbm.at[uidx_v], urows_v, sem))
        for c in copies:
            c.wait()
        pltpu.sync_copy(xrows_v, outx_hbm.at[pl.ds(xbase, xw)])
        pltpu.sync_copy(urows_v, outu_hbm.at[pl.ds(ubase, uw)])

    return gather_k


def _fuse_body(xemb_ref, pu_ref, t_ref, sx_ref, sy_ref, h0_ref, wih_ref,
               whh_ref, bih_ref, bhh_ref, wf_ref, bf_ref, m_ref, out_ref,
               hs_ref, t2v_ref, gi_ref):
    bf16 = jnp.bfloat16

    xe = xemb_ref[...].reshape(S * U, H).astype(bf16)
    gi_all = lax.dot_general(xe, wih_ref[...].astype(bf16),
                             (((1,), (1,)), ((), ())),
                             preferred_element_type=jnp.float32)
    gi_ref[...] = gi_all.reshape(S, U, 3 * H) + bih_ref[...]

    whh = whh_ref[...].astype(bf16)
    bhh = bhh_ref[...]

    def gru_step(i, hprev):
        gi = gi_ref[i]
        gh = lax.dot_general(hprev.astype(bf16), whh,
                             (((1,), (1,)), ((), ())),
                             preferred_element_type=jnp.float32) + bhh
        r = jax.nn.sigmoid(gi[:, :H] + gh[:, :H])
        z = jax.nn.sigmoid(gi[:, H:2 * H] + gh[:, H:2 * H])
        n = jnp.tanh(gi[:, 2 * H:] + r * gh[:, 2 * H:])
        hn = (1.0 - z) * n + z * hprev
        hs_ref[pl.ds(i, 1)] = hn[None]
        return hn

    lax.fori_loop(0, S, gru_step, h0_ref[...], unroll=True)

    tau = t_ref[...][:, :, None]
    ph = tau * wf_ref[...][0] + bf_ref[...][0]
    sp = jnp.sin(ph)
    t2v_ref[...] = sp + m_ref[...][None] * (ph - sp)

    for i in range(S):
        t2vi = t2v_ref[i]
        sxi = sx_ref[pl.ds(i, 1)]
        syi = sy_ref[pl.ds(i, 1)]

        def j_body(j, carry, t2vi=t2vi, sxi=sxi, syi=syi):
            acc, sw = carry
            d = t2v_ref[j] - t2vi
            asq = jnp.sum(d * d, axis=1)
            dx = sx_ref[pl.ds(j, 1)] - sxi
            dy = sy_ref[pl.ds(j, 1)] - syi
            dist = jnp.sqrt(dx * dx + dy * dy + 1e-12)[0]
            w = jnp.exp(-(asq + LAMBDA_LOC * dist)) + 1e-10
            return (acc + w[:, None] * hs_ref[j], sw + w)

        acc, sw = lax.fori_loop(
            0, i + 1, j_body,
            (jnp.zeros((U, H), jnp.float32), jnp.zeros((U,), jnp.float32)))
        out_ref[pl.ds(i, 1), :, :H] = (acc / sw[:, None]).astype(bf16)[None]

    out_ref[:, :, H:] = jnp.broadcast_to(
        pu_ref[...].astype(bf16)[None], (S, U, H))


_BN = 512


def _mm_body(a_ref, w_ref, b_ref, o_ref):
    w = w_ref[...].astype(jnp.bfloat16)
    b = b_ref[...]
    for i in range(S):
        y = lax.dot_general(w, a_ref[i], (((1,), (1,)), ((), ())),
                            preferred_element_type=jnp.float32)
        o_ref[i] = y + b


def _fc_matmul(a, fc_W, fc_b2):
    yt = pl.pallas_call(
        _mm_body,
        grid=(pl.cdiv(V, _BN),),
        in_specs=[
            pl.BlockSpec((S, U, 2 * H), lambda n: (0, 0, 0)),
            pl.BlockSpec((_BN, 2 * H), lambda n: (n, 0)),
            pl.BlockSpec((_BN, 1), lambda n: (n, 0)),
        ],
        out_specs=pl.BlockSpec((S, _BN, U), lambda n: (0, n, 0)),
        out_shape=jax.ShapeDtypeStruct((S, V, U), jnp.float32),
    )(a, fc_W, fc_b2)
    return jnp.transpose(yt, (0, 2, 1))


def kernel(x, t, s, h, active_user, enc_table, gru_Wih, gru_Whh, gru_bih,
           gru_bhh, t2v_w0, t2v_b0, t2v_W, t2v_B, fc_W, fc_b):
    x_rows, u_rows = _make_sc_gather()(
        enc_table, x.reshape(-1).astype(jnp.int32),
        active_user.reshape(-1).astype(jnp.int32))
    x_emb = x_rows.reshape(S, U, H)

    wf = jnp.concatenate([t2v_w0, t2v_W[0]]).reshape(1, H)
    bf = jnp.concatenate([t2v_b0, t2v_B]).reshape(1, H)
    chmask = jnp.zeros((1, H), jnp.float32).at[0, 0].set(1.0)
    out_pu = pl.pallas_call(
        _fuse_body,
        out_shape=jax.ShapeDtypeStruct((S, U, 2 * H), jnp.bfloat16),
        scratch_shapes=[
            pltpu.VMEM((S, U, H), jnp.float32),
            pltpu.VMEM((S, U, H), jnp.float32),
            pltpu.VMEM((S, U, 3 * H), jnp.float32),
        ],
    )(x_emb, u_rows, t, s[:, :, 0], s[:, :, 1], h[0], gru_Wih, gru_Whh,
      gru_bih.reshape(1, 3 * H), gru_bhh.reshape(1, 3 * H), wf, bf, chmask)

    return _fc_matmul(out_pu, fc_W, fc_b.reshape(V, 1))
